# baseline (speedup 1.0000x reference)
import functools

import jax
import jax.numpy as jnp
from jax import lax
from jax.experimental import pallas as pl
from jax.experimental.pallas import tpu as pltpu
from jax.experimental.pallas import tpu_sc as plsc

_NUM_ROW = 13
_NUM_COL = 15
_WIDTH = _NUM_ROW + _NUM_COL
_NTYPES = 118
_LUTPAD = 128
_WPAD = 32

_N = 819200
_NW = 32
_BPW = _N // _NW
_LANES = 16
_CHUNK = 1280
_NCHUNK = _BPW // _CHUNK
_GROUPS = _CHUNK // _LANES


def _sc_body(atoms_hbm, mask_hbm, out_hbm, idx_v, lut_v, obuf_v, sem0, sem1):
    wid = lax.axis_index("s") * 2 + lax.axis_index("c")
    base = wid * _BPW
    pltpu.sync_copy(mask_hbm, lut_v)
    pltpu.sync_copy(atoms_hbm.at[pl.ds(base, _BPW)], idx_v)

    lane = lax.iota(jnp.int32, _LANES)
    lane16 = lane + 16
    sems = (sem0, sem1)

    def fill(c, slot):
        @plsc.parallel_loop(0, _GROUPS, unroll=2)
        def group(g):
            a0 = pl.multiple_of(c * _CHUNK + g * _LANES, _LANES)
            atoms = idx_v[pl.ds(a0, _LANES)]
            w = plsc.load_gather(lut_v, [atoms])
            r0 = pl.multiple_of(g * _LANES, _LANES)
            for i in range(_LANES):
                wb = w.at[jnp.full((_LANES,), i, jnp.int32)].get(
                    mode="promise_in_bounds"
                )
                lo = lax.shift_right_logical(wb, lane) & 1
                hi = lax.shift_right_logical(wb, lane16) & 1
                obuf_v[slot, r0 + i, pl.ds(0, _LANES)] = lo
                obuf_v[slot, r0 + i, pl.ds(16, _LANES)] = hi

    def flush(c, slot):
        r0 = pl.multiple_of(base + c * _CHUNK, _CHUNK)
        pltpu.async_copy(obuf_v.at[slot], out_hbm.at[pl.ds(r0, _CHUNK)], sems[slot])

    def drain(slot):
        pltpu.make_async_copy(
            out_hbm.at[pl.ds(base, _CHUNK)], obuf_v.at[slot], sems[slot]
        ).wait()

    def super_step(cs, _):
        for slot in range(2):
            c = cs * 2 + slot

            @pl.when(cs > 0)
            def _():
                drain(slot)

            fill(c, slot)
            flush(c, slot)
        return 0

    lax.fori_loop(0, _NCHUNK // 2, super_step, 0)
    drain(0)
    drain(1)


def _sc_encode(atom_types, mask_lut):
    mesh = plsc.VectorSubcoreMesh(core_axis_name="c", subcore_axis_name="s")
    run = functools.partial(
        pl.kernel,
        mesh=mesh,
        out_type=jax.ShapeDtypeStruct((_N, _WPAD), jnp.int32),
        scratch_types=[
            pltpu.VMEM((_BPW,), jnp.int32),
            pltpu.VMEM((_LUTPAD,), jnp.int32),
            pltpu.VMEM((2, _CHUNK, _WPAD), jnp.int32),
            pltpu.SemaphoreType.DMA,
            pltpu.SemaphoreType.DMA,
        ],
        compiler_params=pltpu.CompilerParams(
            use_tc_tiling_on_sc=False, needs_layout_passes=False
        ),
    )(_sc_body)
    return run(atom_types, mask_lut)


def kernel(atom_types, reordered_indices):
    rolled = jnp.roll(reordered_indices, 1, axis=0)
    mask = (1 << rolled[:, 0]) | (1 << (_NUM_ROW + rolled[:, 1]))
    mask = jnp.zeros((_LUTPAD,), jnp.int32).at[:_NTYPES].set(mask.astype(jnp.int32))
    return _sc_encode(atom_types, mask)[:, :_WIDTH]


# speedup vs baseline: 8.8040x; 8.8040x over previous
import functools

import jax
import jax.numpy as jnp
from jax import lax
from jax.experimental import pallas as pl
from jax.experimental.pallas import tpu as pltpu
from jax.experimental.pallas import tpu_sc as plsc

_NUM_ROW = 13
_NUM_COL = 15
_WIDTH = _NUM_ROW + _NUM_COL
_NTYPES = 118
_LUTPAD = 128

_N = 819200
_NW = 32
_BPW = _N // _NW
_LANES = 16
_NT = _N // 128
_TPW = _BPW // 128
_TPC = 10
_CHUNK = _TPC * 128
_NCHUNK = _BPW // _CHUNK
_GROUPS = _CHUNK // _LANES


def _sc_body(atoms_hbm, mask_hbm, out_hbm, idx_v, lut_v, tbuf_v, sem0, sem1):
    wid = lax.axis_index("s") * 2 + lax.axis_index("c")
    base = wid * _BPW
    tbase = wid * _TPW
    pltpu.sync_copy(mask_hbm, lut_v)
    pltpu.sync_copy(atoms_hbm.at[pl.ds(base, _BPW)], idx_v)

    sems = (sem0, sem1)

    def fill(c, slot):
        @plsc.parallel_loop(0, _GROUPS, unroll=2)
        def group(g):
            a0 = pl.multiple_of(c * _CHUNK + g * _LANES, _LANES)
            atoms = idx_v[pl.ds(a0, _LANES)]
            w = plsc.load_gather(lut_v, [atoms])
            at = g // 8
            i0 = pl.multiple_of((g % 8) * _LANES, _LANES)
            for j in range(32):
                bit = lax.shift_right_logical(w, j) & 1
                tbuf_v[slot, j // 8, at, j % 8, pl.ds(i0, _LANES)] = bit

    def flush(c, slot):
        t0 = pl.multiple_of(tbase + c * _TPC, _TPC)
        pltpu.async_copy(
            tbuf_v.at[slot], out_hbm.at[:, pl.ds(t0, _TPC)], sems[slot]
        )

    def drain(slot):
        pltpu.make_async_copy(
            out_hbm.at[:, pl.ds(tbase, _TPC)], tbuf_v.at[slot], sems[slot]
        ).wait()

    def super_step(cs, _):
        for slot in range(2):
            c = cs * 2 + slot

            @pl.when(cs > 0)
            def _():
                drain(slot)

            fill(c, slot)
            flush(c, slot)
        return 0

    lax.fori_loop(0, _NCHUNK // 2, super_step, 0)
    drain(0)
    drain(1)


def _sc_encode(atom_types, mask_lut):
    mesh = plsc.VectorSubcoreMesh(core_axis_name="c", subcore_axis_name="s")
    run = functools.partial(
        pl.kernel,
        mesh=mesh,
        out_type=jax.ShapeDtypeStruct((4, _NT, 8, 128), jnp.int32),
        scratch_types=[
            pltpu.VMEM((_BPW,), jnp.int32),
            pltpu.VMEM((_LUTPAD,), jnp.int32),
            pltpu.VMEM((2, 4, _TPC, 8, 128), jnp.int32),
            pltpu.SemaphoreType.DMA,
            pltpu.SemaphoreType.DMA,
        ],
        compiler_params=pltpu.CompilerParams(
            use_tc_tiling_on_sc=False, needs_layout_passes=False
        ),
    )(_sc_body)
    return run(atom_types, mask_lut)


def kernel(atom_types, reordered_indices):
    rolled = jnp.roll(reordered_indices, 1, axis=0)
    mask = (1 << rolled[:, 0]) | (1 << (_NUM_ROW + rolled[:, 1]))
    mask = jnp.zeros((_LUTPAD,), jnp.int32).at[:_NTYPES].set(mask.astype(jnp.int32))
    out4 = _sc_encode(atom_types, mask)
    return out4.transpose(1, 3, 0, 2).reshape(_N, 32)[:, :_WIDTH]


# speedup vs baseline: 9.4772x; 1.0765x over previous
import functools

import jax
import jax.numpy as jnp
from jax import lax
from jax.experimental import pallas as pl
from jax.experimental.pallas import tpu as pltpu
from jax.experimental.pallas import tpu_sc as plsc

_NUM_ROW = 13
_NUM_COL = 15
_WIDTH = _NUM_ROW + _NUM_COL
_NTYPES = 118
_LUTPAD = 128

_N = 819200
_NW = 32
_BPW = _N // _NW
_LANES = 16
_NT = _N // 128
_TPW = _BPW // 128
_TPC = 10
_CHUNK = _TPC * 128
_NCHUNK = _BPW // _CHUNK
_GROUPS = _CHUNK // _LANES


def _sc_body(atoms_hbm, mask_hbm, out_hbm, idx_v, lut_v, tbuf_v, sem0, sem1):
    wid = lax.axis_index("s") * 2 + lax.axis_index("c")
    base = wid * _BPW
    tbase = wid * _TPW
    pltpu.sync_copy(mask_hbm, lut_v)
    pltpu.sync_copy(atoms_hbm.at[pl.ds(base, _BPW)], idx_v)

    sems = (sem0, sem1)

    def fill(c, slot):
        @plsc.parallel_loop(0, _GROUPS, unroll=2)
        def group(g):
            a0 = pl.multiple_of(c * _CHUNK + g * _LANES, _LANES)
            atoms = idx_v[pl.ds(a0, _LANES)]
            w = plsc.load_gather(lut_v, [atoms])
            at = g // 8
            i0 = pl.multiple_of((g % 8) * _LANES, _LANES)
            for j in range(_WIDTH):
                bit = lax.shift_right_logical(w, j) & 1
                tbuf_v[slot, j // 8, at, j % 8, pl.ds(i0, _LANES)] = bit

    def flush(c, slot):
        t0 = pl.multiple_of(tbase + c * _TPC, _TPC)
        pltpu.async_copy(
            tbuf_v.at[slot, pl.ds(0, 3)],
            out_hbm.at[pl.ds(0, 3), pl.ds(t0, _TPC)],
            sems[slot],
        )
        pltpu.async_copy(
            tbuf_v.at[slot, 3, :, pl.ds(0, 4)],
            out_hbm.at[3, pl.ds(t0, _TPC), pl.ds(0, 4)],
            sems[slot],
        )

    def drain(slot):
        pltpu.make_async_copy(
            out_hbm.at[pl.ds(0, 3), pl.ds(tbase, _TPC)],
            tbuf_v.at[slot, pl.ds(0, 3)],
            sems[slot],
        ).wait()
        pltpu.make_async_copy(
            out_hbm.at[3, pl.ds(tbase, _TPC), pl.ds(0, 4)],
            tbuf_v.at[slot, 3, :, pl.ds(0, 4)],
            sems[slot],
        ).wait()

    def super_step(cs, _):
        for slot in range(2):
            c = cs * 2 + slot

            @pl.when(cs > 0)
            def _():
                drain(slot)

            fill(c, slot)
            flush(c, slot)
        return 0

    lax.fori_loop(0, _NCHUNK // 2, super_step, 0)
    drain(0)
    drain(1)


def _sc_encode(atom_types, mask_lut):
    mesh = plsc.VectorSubcoreMesh(core_axis_name="c", subcore_axis_name="s")
    run = functools.partial(
        pl.kernel,
        mesh=mesh,
        out_type=jax.ShapeDtypeStruct((4, _NT, 8, 128), jnp.int32),
        scratch_types=[
            pltpu.VMEM((_BPW,), jnp.int32),
            pltpu.VMEM((_LUTPAD,), jnp.int32),
            pltpu.VMEM((2, 4, _TPC, 8, 128), jnp.int32),
            pltpu.SemaphoreType.DMA,
            pltpu.SemaphoreType.DMA,
        ],
        compiler_params=pltpu.CompilerParams(
            use_tc_tiling_on_sc=False, needs_layout_passes=False
        ),
    )(_sc_body)
    return run(atom_types, mask_lut)


def kernel(atom_types, reordered_indices):
    rolled = jnp.roll(reordered_indices, 1, axis=0)
    mask = (1 << rolled[:, 0]) | (1 << (_NUM_ROW + rolled[:, 1]))
    mask = jnp.zeros((_LUTPAD,), jnp.int32).at[:_NTYPES].set(mask.astype(jnp.int32))
    out4 = _sc_encode(atom_types, mask)
    return out4.transpose(1, 3, 0, 2).reshape(_N, 32)[:, :_WIDTH]
